# Initial kernel scaffold; baseline (speedup 1.0000x reference)
import jax
import jax.numpy as jnp
from jax.experimental import pallas as pl


def kernel(node_tensors, edge_tensors, edge_index, Wnq_w, Wnq_b, Wnk_w, Wnk_b, Wnv_w, Wnv_b, Weq_w, Weq_b, Wek_w, Wek_b, Wev_w, Wev_b, Wo_w, Wo_b):
    raise NotImplementedError("write your pallas kernel here")


# speedup vs baseline: 17.1305x; 17.1305x over previous
import functools
import math

import jax
import jax.numpy as jnp
from jax import lax
from jax.experimental import pallas as pl
from jax.experimental.pallas import tpu as pltpu
from jax.experimental.pallas import tpu_sc as plsc

N = 10000
E = 320000
ND = 128
ED = 16
H = 8
D = 16
SCALE = 1.0 / math.sqrt(D)

NB = 1000
EB = 4000
G = 80
NW = 32
EPW = E // NW
NGROUPS = EPW // G


def _tables_body(node_ref, wnq_ref, wnk_ref, wnv_ref, bkd_ref, bqd_ref,
                 bq_ref, bk_ref, bv_ref, a_ref, b_ref, v_ref):
    x = node_ref[...]
    dn = (((1,), (1,)), ((), ()))
    at = lax.dot_general(x, wnq_ref[...], dn,
                         preferred_element_type=jnp.float32) + bq_ref[...]
    bt = lax.dot_general(x, wnk_ref[...], dn,
                         preferred_element_type=jnp.float32) + bk_ref[...]
    vt = lax.dot_general(x, wnv_ref[...], dn,
                         preferred_element_type=jnp.float32) + bv_ref[...]
    ap = jnp.dot(at, bkd_ref[...], preferred_element_type=jnp.float32)
    bp = jnp.dot(bt, bqd_ref[...], preferred_element_type=jnp.float32)
    a_ref[:, :ND] = at
    a_ref[:, ND:] = ap
    b_ref[:, :ND] = bt
    b_ref[:, ND:] = bp
    v_ref[...] = vt


def _make_tables(node, wnq, wnk, wnv, bkd, bqd, bq, bk, bv):
    full = lambda shape: pl.BlockSpec(shape, lambda i: (0,) * len(shape))
    return pl.pallas_call(
        _tables_body,
        grid=(N // NB,),
        in_specs=[
            pl.BlockSpec((NB, ND), lambda i: (i, 0)),
            full((ND, ND)), full((ND, ND)), full((ND, ND)),
            full((ND, ND)), full((ND, ND)),
            full((1, ND)), full((1, ND)), full((1, ND)),
        ],
        out_specs=[
            pl.BlockSpec((NB, 2 * ND), lambda i: (i, 0)),
            pl.BlockSpec((NB, 2 * ND), lambda i: (i, 0)),
            pl.BlockSpec((NB, ND), lambda i: (i, 0)),
        ],
        out_shape=[
            jax.ShapeDtypeStruct((N, 2 * ND), jnp.float32),
            jax.ShapeDtypeStruct((N, 2 * ND), jnp.float32),
            jax.ShapeDtypeStruct((N, ND), jnp.float32),
        ],
    )(node, wnq, wnk, wnv, bkd, bqd, bq, bk, bv)


def _t1_body(edge_ref, weq_ref, wek_ref, ones_ref, t1_ref):
    x = edge_ref[...]
    dn = (((1,), (1,)), ((), ()))
    eq = lax.dot_general(x, weq_ref[...], dn,
                         preferred_element_type=jnp.float32)
    ek = lax.dot_general(x, wek_ref[...], dn,
                         preferred_element_type=jnp.float32)
    t1_ref[...] = jnp.dot(eq * ek, ones_ref[...],
                          preferred_element_type=jnp.float32)


def _make_t1(edge, weq, wek, ones8):
    return pl.pallas_call(
        _t1_body,
        grid=(E // EB,),
        in_specs=[
            pl.BlockSpec((EB, ED), lambda i: (i, 0)),
            pl.BlockSpec((ND, ED), lambda i: (0, 0)),
            pl.BlockSpec((ND, ED), lambda i: (0, 0)),
            pl.BlockSpec((ND, H), lambda i: (0, 0)),
        ],
        out_specs=pl.BlockSpec((EB, H), lambda i: (i, 0)),
        out_shape=jax.ShapeDtypeStruct((E, H), jnp.float32),
    )(edge, weq, wek, ones8)


def _sc1_body(a_hbm, b_hbm, edge_hbm, t1_hbm, src_hbm, dst_hbm,
              ex_hbm,
              src_v, dst_v, a_rows, b_rows, edge_v, t1_v, ex_v, sem):
    c = lax.axis_index("c")
    s = lax.axis_index("s")
    wid = c * 16 + s
    iot = jnp.arange(16, dtype=jnp.int32)

    def group(g, _):
        off = wid * EPW + g * G
        pltpu.sync_copy(src_hbm.at[pl.ds(off, G)], src_v)
        pltpu.sync_copy(dst_hbm.at[pl.ds(off, G)], dst_v)
        pltpu.sync_copy(edge_hbm.at[pl.ds(off, G), :], edge_v)
        pltpu.sync_copy(t1_hbm.at[pl.ds(off, G), :], t1_v)
        pltpu.async_copy(a_hbm.at[src_v], a_rows, sem).wait()
        pltpu.async_copy(b_hbm.at[dst_v], b_rows, sem).wait()

        def edge_body(e, _):
            ev = edge_v[e, :]
            acc = jnp.zeros((16,), jnp.float32)
            for h in range(H):
                qh = a_rows[e, pl.ds(h * 16, 16)]
                kh = b_rows[e, pl.ds(h * 16, 16)]
                ph = (a_rows[e, pl.ds(ND + h * 16, 16)]
                      + b_rows[e, pl.ds(ND + h * 16, 16)])
                sh = jnp.sum(qh * kh + ph * ev)
                acc = acc + jnp.where(iot == h, sh, jnp.float32(0.0))
            esp = jnp.full((16,), e, jnp.int32)
            t1g = plsc.load_gather(t1_v, [esp, iot & 7])
            exv = jnp.exp((acc + t1g) * SCALE)
            ex_v[e, :] = exv
            return 0
        lax.fori_loop(0, G, edge_body, 0)

        pltpu.sync_copy(ex_v, ex_hbm.at[pl.ds(off, G), :])
        return 0
    lax.fori_loop(0, NGROUPS, group, 0)


def _run_sc1(a, b, edge, t1, src, dst):
    mesh = plsc.VectorSubcoreMesh(core_axis_name="c", subcore_axis_name="s")
    kern = functools.partial(
        pl.kernel,
        mesh=mesh,
        out_type=jax.ShapeDtypeStruct((E, 16), jnp.float32),
        scratch_types=[
            pltpu.VMEM((G,), jnp.int32),
            pltpu.VMEM((G,), jnp.int32),
            pltpu.VMEM((G, 2 * ND), jnp.float32),
            pltpu.VMEM((G, 2 * ND), jnp.float32),
            pltpu.VMEM((G, ED), jnp.float32),
            pltpu.VMEM((G, H), jnp.float32),
            pltpu.VMEM((G, 16), jnp.float32),
            pltpu.SemaphoreType.DMA,
        ],
        compiler_params=pltpu.CompilerParams(
            needs_layout_passes=False, use_tc_tiling_on_sc=False),
    )(_sc1_body)
    return kern(a, b, edge, t1, src, dst)


def _sc2_body(edge_hbm, ex_hbm, src_hbm,
              pssum_hbm, pu_hbm,
              src_v, edge_v, ex_v, urows, uidx, zbuf, u_sh, ssum_sh, sem):
    c = lax.axis_index("c")
    s = lax.axis_index("s")
    wid = c * 16 + s
    iot = jnp.arange(16, dtype=jnp.int32)

    def zrow(i, _):
        zbuf[i, :] = jnp.zeros((16,), jnp.float32)
        return 0
    lax.fori_loop(0, 200, zrow, 0)
    for k in range(25):
        pltpu.sync_copy(zbuf, u_sh.at[pl.ds((s * 25 + k) * 200, 200), :])
    @pl.when(s < 10)
    def _():
        for k in range(5):
            pltpu.sync_copy(zbuf,
                            ssum_sh.at[pl.ds((s * 5 + k) * 200, 200), :])
    plsc.subcore_barrier()

    def group(g, _):
        off = wid * EPW + g * G
        pltpu.sync_copy(src_hbm.at[pl.ds(off, G)], src_v)
        pltpu.sync_copy(edge_hbm.at[pl.ds(off, G), :], edge_v)
        pltpu.sync_copy(ex_hbm.at[pl.ds(off, G), :], ex_v)

        for v in range(40):
            r = iot + v * 16
            sv = plsc.load_gather(src_v, [r >> 3])
            uidx[v // 8, pl.ds((v % 8) * 16, 16)] = sv * 8 + (r & 7)

        def edge_body(e, _):
            ev = edge_v[e, :]
            exv = ex_v[e, :]
            for h in range(H):
                urows[e * 8 + h, :] = exv[h] * ev
            return 0
        lax.fori_loop(0, G, edge_body, 0)

        pltpu.sync_copy(ex_v, ssum_sh.at[src_v], add=True)
        for k in range(5):
            pltpu.sync_copy(urows.at[pl.ds(k * 128, 128), :],
                            u_sh.at[uidx.at[k]], add=True)
        return 0
    lax.fori_loop(0, NGROUPS, group, 0)
    plsc.subcore_barrier()

    for k in range(5):
        rs = (s * 5 + k) * 1000
        pltpu.sync_copy(u_sh.at[pl.ds(rs, 1000), :],
                        pu_hbm.at[c, pl.ds(rs, 1000), :])
    @pl.when(s < 10)
    def _():
        pltpu.sync_copy(ssum_sh.at[pl.ds(s * 1000, 1000), :],
                        pssum_hbm.at[c, pl.ds(s * 1000, 1000), :])


def _run_sc2(edge, ex, src):
    mesh = plsc.VectorSubcoreMesh(core_axis_name="c", subcore_axis_name="s")
    kern = functools.partial(
        pl.kernel,
        mesh=mesh,
        out_type=[
            jax.ShapeDtypeStruct((2, N, 16), jnp.float32),
            jax.ShapeDtypeStruct((2, N * H, 16), jnp.float32),
        ],
        scratch_types=[
            pltpu.VMEM((G,), jnp.int32),
            pltpu.VMEM((G, ED), jnp.float32),
            pltpu.VMEM((G, 16), jnp.float32),
            pltpu.VMEM((G * H, 16), jnp.float32),
            pltpu.VMEM((5, 128), jnp.int32),
            pltpu.VMEM((200, 16), jnp.float32),
            pltpu.VMEM_SHARED((N * H, 16), jnp.float32),
            pltpu.VMEM_SHARED((N, 16), jnp.float32),
            pltpu.SemaphoreType.DMA,
        ],
        compiler_params=pltpu.CompilerParams(
            needs_layout_passes=False, use_tc_tiling_on_sc=False),
    )(_sc2_body)
    return kern(edge, ex, src)


def _sc3_body(v_hbm, ex_hbm, src_hbm, dst_hbm, pw2_hbm,
              src_v, dst_v, v_rows, ex_v, w2rows, uidx, zbuf, w2_sh, sem):
    c = lax.axis_index("c")
    s = lax.axis_index("s")
    wid = c * 16 + s
    iot = jnp.arange(16, dtype=jnp.int32)

    def zrow(i, _):
        zbuf[i, :] = jnp.zeros((16,), jnp.float32)
        return 0
    lax.fori_loop(0, 200, zrow, 0)
    for k in range(25):
        pltpu.sync_copy(zbuf, w2_sh.at[pl.ds((s * 25 + k) * 200, 200), :])
    plsc.subcore_barrier()

    def group(g, _):
        off = wid * EPW + g * G
        pltpu.sync_copy(src_hbm.at[pl.ds(off, G)], src_v)
        pltpu.sync_copy(dst_hbm.at[pl.ds(off, G)], dst_v)
        pltpu.sync_copy(ex_hbm.at[pl.ds(off, G), :], ex_v)
        pltpu.async_copy(v_hbm.at[dst_v], v_rows, sem).wait()

        for v in range(40):
            r = iot + v * 16
            sv = plsc.load_gather(src_v, [r >> 3])
            uidx[v // 8, pl.ds((v % 8) * 16, 16)] = sv * 8 + (r & 7)

        def edge_body(e, _):
            exv = ex_v[e, :]
            for h in range(H):
                w2rows[e * 8 + h, :] = exv[h] * v_rows[e, pl.ds(h * 16, 16)]
            return 0
        lax.fori_loop(0, G, edge_body, 0)

        for k in range(5):
            pltpu.sync_copy(w2rows.at[pl.ds(k * 128, 128), :],
                            w2_sh.at[uidx.at[k]], add=True)
        return 0
    lax.fori_loop(0, NGROUPS, group, 0)
    plsc.subcore_barrier()

    for k in range(5):
        rs = (s * 5 + k) * 1000
        pltpu.sync_copy(w2_sh.at[pl.ds(rs, 1000), :],
                        pw2_hbm.at[c, pl.ds(rs, 1000), :])


def _run_sc3(v, ex, src, dst):
    mesh = plsc.VectorSubcoreMesh(core_axis_name="c", subcore_axis_name="s")
    kern = functools.partial(
        pl.kernel,
        mesh=mesh,
        out_type=jax.ShapeDtypeStruct((2, N * H, 16), jnp.float32),
        scratch_types=[
            pltpu.VMEM((G,), jnp.int32),
            pltpu.VMEM((G,), jnp.int32),
            pltpu.VMEM((G, ND), jnp.float32),
            pltpu.VMEM((G, 16), jnp.float32),
            pltpu.VMEM((G * H, 16), jnp.float32),
            pltpu.VMEM((5, 128), jnp.int32),
            pltpu.VMEM((200, 16), jnp.float32),
            pltpu.VMEM_SHARED((N * H, 16), jnp.float32),
            pltpu.SemaphoreType.DMA,
        ],
        compiler_params=pltpu.CompilerParams(
            needs_layout_passes=False, use_tc_tiling_on_sc=False),
    )(_sc3_body)
    return kern(v, ex, src, dst)


def _combine_body(pu_ref, pw2_ref, ps_ref, bvdt_ref, emat_ref, wo_ref,
                  wob_ref, out_ref):
    u = pu_ref[0] + pu_ref[1]
    w2 = pw2_ref[0] + pw2_ref[1]
    ssum = (ps_ref[0] + ps_ref[1])[:, 0:H]
    recip = 1.0 / (ssum + 1e-16)
    scl = jnp.dot(recip, emat_ref[...], preferred_element_type=jnp.float32)
    agg = (jnp.dot(u, bvdt_ref[...], preferred_element_type=jnp.float32)
           + w2) * scl
    dn = (((1,), (1,)), ((), ()))
    out_ref[...] = lax.dot_general(
        agg, wo_ref[...], dn, preferred_element_type=jnp.float32
    ) + wob_ref[...]


def _combine(pu, pw2, pssum, bvdt, emat, wo, wob):
    return pl.pallas_call(
        _combine_body,
        grid=(N // NB,),
        in_specs=[
            pl.BlockSpec((2, NB, ND), lambda i: (0, i, 0)),
            pl.BlockSpec((2, NB, ND), lambda i: (0, i, 0)),
            pl.BlockSpec((2, NB, 16), lambda i: (0, i, 0)),
            pl.BlockSpec((ND, ND), lambda i: (0, 0)),
            pl.BlockSpec((H, ND), lambda i: (0, 0)),
            pl.BlockSpec((ND, ND), lambda i: (0, 0)),
            pl.BlockSpec((1, ND), lambda i: (0, 0)),
        ],
        out_specs=pl.BlockSpec((NB, ND), lambda i: (i, 0)),
        out_shape=jax.ShapeDtypeStruct((N, ND), jnp.float32),
    )(pu, pw2, pssum, bvdt, emat, wo, wob)


def _blockdiag(w, transpose=False):
    m = jnp.zeros((ND, ND), jnp.float32)
    for h in range(H):
        blk = w[h * 16:(h + 1) * 16, :]
        if transpose:
            blk = blk.T
        m = m.at[h * 16:(h + 1) * 16, h * 16:(h + 1) * 16].set(blk)
    return m


def kernel(node_tensors, edge_tensors, edge_index, Wnq_w, Wnq_b, Wnk_w,
           Wnk_b, Wnv_w, Wnv_b, Weq_w, Weq_b, Wek_w, Wek_b, Wev_w, Wev_b,
           Wo_w, Wo_b):
    src = edge_index[0]
    dst = edge_index[1]
    bq = (Wnq_b + Weq_b).reshape(1, ND)
    bk = (Wnk_b + Wek_b).reshape(1, ND)
    bv = (Wnv_b + Wev_b).reshape(1, ND)
    bkd = _blockdiag(Wek_w)
    bqd = _blockdiag(Weq_w)
    bvdt = _blockdiag(Wev_w, transpose=True)
    ones8 = jnp.zeros((ND, H), jnp.float32)
    for h in range(H):
        ones8 = ones8.at[h * 16:(h + 1) * 16, h].set(1.0)
    emat = ones8.T

    a, b, v = _make_tables(node_tensors, Wnq_w, Wnk_w, Wnv_w, bkd, bqd,
                           bq, bk, bv)
    t1 = _make_t1(edge_tensors, Weq_w, Wek_w, ones8)
    ex = _run_sc1(a, b, edge_tensors, t1, src, dst)
    pssum, pu = _run_sc2(edge_tensors, ex, src)
    pw2 = _run_sc3(v, ex, src, dst)
    pu_r = pu.reshape(2, N, ND)
    pw2_r = pw2.reshape(2, N, ND)
    return _combine(pu_r, pw2_r, pssum, bvdt, emat, Wo_w,
                    Wo_b.reshape(1, ND))
